# baseline (speedup 1.0000x reference)
import functools

import jax
import jax.numpy as jnp
from jax import lax
from jax.experimental import pallas as pl
from jax.experimental.pallas import tpu as pltpu
from jax.experimental.pallas import tpu_sc as plsc

_NC = 2
_NS = 16
_NW = _NC * _NS


@functools.lru_cache(maxsize=None)
def _make_sc(seq_len, batch, embed_dim, dtype):
    rows_per_w = seq_len // _NW
    chunk = min(rows_per_w, 32)
    n_chunks = rows_per_w // chunk
    n_buf = 3
    mesh = plsc.VectorSubcoreMesh(core_axis_name="c", subcore_axis_name="s")

    @functools.partial(
        pl.kernel,
        mesh=mesh,
        out_type=jax.ShapeDtypeStruct((seq_len, batch, embed_dim), dtype),
        scratch_types=(
            [pltpu.VMEM((chunk, embed_dim), dtype) for _ in range(n_buf)]
            + [pltpu.SemaphoreType.DMA for _ in range(n_buf)]
            + [pltpu.SemaphoreType.DMA for _ in range(n_buf)]
        ),
    )
    def k(table_hbm, out_hbm, *scratch):
        bufs = scratch[:n_buf]
        lsem = scratch[n_buf : 2 * n_buf]
        wsem = scratch[2 * n_buf : 3 * n_buf]
        wid = lax.axis_index("s") * _NC + lax.axis_index("c")
        base = wid * rows_per_w

        def load(c):
            s0 = base + c * chunk
            d = pltpu.make_async_copy(
                table_hbm.at[pl.ds(s0, chunk)], bufs[c % n_buf], lsem[c % n_buf]
            )
            d.start()
            return d

        def writes(c):
            s0 = base + c * chunk
            ds = []
            for b in range(batch):
                d = pltpu.make_async_copy(
                    bufs[c % n_buf],
                    out_hbm.at[pl.ds(s0, chunk), b],
                    wsem[c % n_buf],
                )
                d.start()
                ds.append(d)
            return ds

        pending_w = [None] * n_buf
        lds = {0: load(0)}
        for c in range(n_chunks):
            nxt = c + 1
            if nxt < n_chunks:
                nb = nxt % n_buf
                if pending_w[nb] is not None:
                    for d in pending_w[nb]:
                        d.wait()
                    pending_w[nb] = None
                lds[nxt] = load(nxt)
            lds.pop(c).wait()
            pending_w[c % n_buf] = writes(c)
        for ds in pending_w:
            if ds is not None:
                for d in ds:
                    d.wait()

    return k


def kernel(x, pos_embedding):
    seq_len, batch = x.shape
    max_len, embed_dim = pos_embedding.shape
    k = _make_sc(seq_len, batch, embed_dim, pos_embedding.dtype)
    return k(pos_embedding)


# speedup vs baseline: 1.0901x; 1.0901x over previous
import functools

import jax
import jax.numpy as jnp
from jax import lax
from jax.experimental import pallas as pl
from jax.experimental.pallas import tpu as pltpu
from jax.experimental.pallas import tpu_sc as plsc

_NC = 2
_NS = 16
_NW = _NC * _NS


@functools.lru_cache(maxsize=None)
def _make_sc(seq_len, batch, embed_dim, dtype):
    rows_per_w = seq_len // _NW
    chunk = min(rows_per_w, 128)
    n_chunks = rows_per_w // chunk
    n_buf = 1
    mesh = plsc.VectorSubcoreMesh(core_axis_name="c", subcore_axis_name="s")

    @functools.partial(
        pl.kernel,
        mesh=mesh,
        out_type=jax.ShapeDtypeStruct((seq_len, batch, embed_dim), dtype),
        scratch_types=(
            [pltpu.VMEM((chunk, embed_dim), dtype) for _ in range(n_buf)]
            + [pltpu.SemaphoreType.DMA for _ in range(n_buf)]
            + [pltpu.SemaphoreType.DMA for _ in range(n_buf)]
        ),
    )
    def k(table_hbm, out_hbm, *scratch):
        bufs = scratch[:n_buf]
        lsem = scratch[n_buf : 2 * n_buf]
        wsem = scratch[2 * n_buf : 3 * n_buf]
        wid = lax.axis_index("s") * _NC + lax.axis_index("c")
        base = wid * rows_per_w

        def load(c):
            s0 = base + c * chunk
            d = pltpu.make_async_copy(
                table_hbm.at[pl.ds(s0, chunk)], bufs[c % n_buf], lsem[c % n_buf]
            )
            d.start()
            return d

        def writes(c):
            s0 = base + c * chunk
            ds = []
            for b in range(batch):
                d = pltpu.make_async_copy(
                    bufs[c % n_buf],
                    out_hbm.at[pl.ds(s0, chunk), b],
                    wsem[c % n_buf],
                )
                d.start()
                ds.append(d)
            return ds

        for c in range(n_chunks):
            load(c).wait()
            for d in writes(c):
                d.wait()

    return k


def kernel(x, pos_embedding):
    seq_len, batch = x.shape
    max_len, embed_dim = pos_embedding.shape
    k = _make_sc(seq_len, batch, embed_dim, pos_embedding.dtype)
    return k(pos_embedding)


# speedup vs baseline: 1.0901x; 1.0001x over previous
import functools

import jax
import jax.numpy as jnp
from jax import lax
from jax.experimental import pallas as pl
from jax.experimental.pallas import tpu as pltpu
from jax.experimental.pallas import tpu_sc as plsc

_NC = 2
_NS = 16
_NW = _NC * _NS


@functools.lru_cache(maxsize=None)
def _make_sc(seq_len, batch, embed_dim, dtype):
    rows_per_w = seq_len // _NW
    if rows_per_w % 8 == 0 and rows_per_w >= 64:
        q = rows_per_w // 8
        chunks = [3 * q, 2 * q, 3 * q]
        bufsizes = [3 * q, 2 * q]
        bufidx = [0, 1, 0]
    else:
        chunks = [rows_per_w]
        bufsizes = [rows_per_w]
        bufidx = [0]
    starts = [sum(chunks[:i]) for i in range(len(chunks))]
    mesh = plsc.VectorSubcoreMesh(core_axis_name="c", subcore_axis_name="s")

    @functools.partial(
        pl.kernel,
        mesh=mesh,
        out_type=jax.ShapeDtypeStruct((seq_len, batch, embed_dim), dtype),
        scratch_types=(
            [pltpu.VMEM((n, embed_dim), dtype) for n in bufsizes]
            + [pltpu.SemaphoreType.DMA for _ in bufsizes]
            + [pltpu.SemaphoreType.DMA for _ in bufsizes]
        ),
    )
    def k(table_hbm, out_hbm, *scratch):
        nb = len(bufsizes)
        bufs = scratch[:nb]
        lsem = scratch[nb : 2 * nb]
        wsem = scratch[2 * nb : 3 * nb]
        wid = lax.axis_index("s") * _NC + lax.axis_index("c")
        base = wid * rows_per_w

        def load(c):
            j = bufidx[c]
            d = pltpu.make_async_copy(
                table_hbm.at[pl.ds(base + starts[c], chunks[c])],
                bufs[j].at[pl.ds(0, chunks[c])],
                lsem[j],
            )
            d.start()
            return d

        def writes(c):
            j = bufidx[c]
            ds = []
            for b in range(batch):
                d = pltpu.make_async_copy(
                    bufs[j].at[pl.ds(0, chunks[c])],
                    out_hbm.at[pl.ds(base + starts[c], chunks[c]), b],
                    wsem[j],
                )
                d.start()
                ds.append(d)
            return ds

        pending = {}
        lds = {}
        n = len(chunks)
        for c in range(min(2, n)):
            lds[c] = load(c)
        for c in range(n):
            lds.pop(c).wait()
            pending[c] = writes(c)
            nxt = c + 2
            if nxt < n:
                prev = nxt - 2
                for d in pending.pop(prev):
                    d.wait()
                lds[nxt] = load(nxt)
        for ds in pending.values():
            for d in ds:
                d.wait()

    return k


def kernel(x, pos_embedding):
    seq_len, batch = x.shape
    max_len, embed_dim = pos_embedding.shape
    k = _make_sc(seq_len, batch, embed_dim, pos_embedding.dtype)
    return k(pos_embedding)


# speedup vs baseline: 1.5431x; 1.4155x over previous
import jax
import jax.numpy as jnp
from jax.experimental import pallas as pl


def _body(emb_ref, out_ref):
    emb = emb_ref[...]
    out_ref[...] = jnp.broadcast_to(emb[:, None, :], out_ref.shape)


def kernel(x, pos_embedding):
    seq_len, batch = x.shape
    max_len, embed_dim = pos_embedding.shape
    blk = 512
    out = pl.pallas_call(
        _body,
        grid=(seq_len // blk,),
        in_specs=[pl.BlockSpec((blk, embed_dim), lambda i: (i, 0))],
        out_specs=pl.BlockSpec((blk, batch, embed_dim), lambda i: (i, 0, 0)),
        out_shape=jax.ShapeDtypeStruct(
            (seq_len, batch, embed_dim), pos_embedding.dtype
        ),
    )(pos_embedding)
    return out
